# baseline (speedup 1.0000x reference)
import functools

import jax
import jax.numpy as jnp
from jax import lax
from jax.experimental import pallas as pl
from jax.experimental.pallas import tpu as pltpu
from jax.experimental.pallas import tpu_sc as plsc

_NC, _NS = 2, 16
_NW = _NC * _NS


@functools.partial(jax.jit, static_argnames=("B", "D", "C"))
def _sc_gather(table, idx, B, D, C):
    b_per_w = B // _NW
    nchunks = b_per_w // C
    mesh = plsc.VectorSubcoreMesh(core_axis_name="c", subcore_axis_name="s")

    @functools.partial(
        pl.kernel,
        mesh=mesh,
        out_type=jax.ShapeDtypeStruct((B, D), table.dtype),
        compiler_params=pltpu.CompilerParams(use_tc_tiling_on_sc=False),
        scratch_types=[
            pltpu.VMEM((C,), jnp.int32),
            pltpu.VMEM((C, D), table.dtype),
            pltpu.SemaphoreType.DMA,
        ],
    )
    def k(table_hbm, idx_hbm, out_hbm, idx_v, rows_v, sem):
        wid = lax.axis_index("s") * _NC + lax.axis_index("c")
        base = wid * b_per_w

        @pl.loop(0, nchunks)
        def _(i):
            off = base + i * C
            pltpu.sync_copy(idx_hbm.at[pl.ds(off, C)], idx_v)
            pltpu.async_copy(table_hbm.at[idx_v], rows_v, sem).wait()
            pltpu.sync_copy(rows_v, out_hbm.at[pl.ds(off, C)])

    return k(table, idx)


def kernel(input, weight):
    Bb, Ll = input.shape
    D = weight.shape[1]
    B = Bb * Ll
    flat = input.reshape(B).astype(jnp.int32)
    out = _sc_gather(weight, flat, B=B, D=D, C=1600)
    return out.reshape(Bb, Ll, D)


# speedup vs baseline: 1.0011x; 1.0011x over previous
import functools

import jax
import jax.numpy as jnp
from jax import lax
from jax.experimental import pallas as pl
from jax.experimental.pallas import tpu as pltpu
from jax.experimental.pallas import tpu_sc as plsc

_NC, _NS = 2, 16
_NW = _NC * _NS


@functools.partial(jax.jit, static_argnames=("B", "D", "C"))
def _sc_gather(table, idx, B, D, C):
    b_per_w = B // _NW
    nchunks = b_per_w // C
    mesh = plsc.VectorSubcoreMesh(core_axis_name="c", subcore_axis_name="s")

    @functools.partial(
        pl.kernel,
        mesh=mesh,
        out_type=jax.ShapeDtypeStruct((B, D), table.dtype),
        compiler_params=pltpu.CompilerParams(use_tc_tiling_on_sc=False),
        scratch_types=[
            pltpu.VMEM((b_per_w,), jnp.int32),
            pltpu.VMEM((C, D), table.dtype),
            pltpu.VMEM((C, D), table.dtype),
            pltpu.SemaphoreType.DMA,
            pltpu.SemaphoreType.DMA,
            pltpu.SemaphoreType.DMA,
            pltpu.SemaphoreType.DMA,
        ],
    )
    def k(table_hbm, idx_hbm, out_hbm, idx_v, rows0, rows1, g0, g1, w0, w1):
        wid = lax.axis_index("s") * _NC + lax.axis_index("c")
        base = wid * b_per_w
        pltpu.sync_copy(idx_hbm.at[pl.ds(base, b_per_w)], idx_v)
        bufs, gs, ws = (rows0, rows1), (g0, g1), (w0, w1)

        def gather_start(b, c):
            pltpu.async_copy(table_hbm.at[idx_v.at[pl.ds(c * C, C)]], bufs[b], gs[b])

        def gather_wait(b):
            pltpu.make_async_copy(
                table_hbm.at[idx_v.at[pl.ds(0, C)]], bufs[b], gs[b]
            ).wait()

        def wb_start(b, c):
            pltpu.async_copy(bufs[b], out_hbm.at[pl.ds(base + c * C, C)], ws[b])

        def wb_wait(b):
            pltpu.make_async_copy(
                bufs[b], out_hbm.at[pl.ds(base, C)], ws[b]
            ).wait()

        gather_start(0, 0)
        gather_start(1, 1)

        @pl.loop(0, nchunks - 2, step=2)
        def _(g):
            for b in range(2):
                c = g + b
                gather_wait(b)
                wb_start(b, c)
                wb_wait(b)
                gather_start(b, c + 2)

        for b in range(2):
            gather_wait(b)
            wb_start(b, nchunks - 2 + b)
            wb_wait(b)

    return k(table, idx)


def kernel(input, weight):
    Bb, Ll = input.shape
    D = weight.shape[1]
    B = Bb * Ll
    flat = input.reshape(B).astype(jnp.int32)
    out = _sc_gather(weight, flat, B=B, D=D, C=800)
    return out.reshape(Bb, Ll, D)


# speedup vs baseline: 1.1170x; 1.1157x over previous
import functools

import jax
import jax.numpy as jnp
from jax import lax
from jax.experimental import pallas as pl
from jax.experimental.pallas import tpu as pltpu
from jax.experimental.pallas import tpu_sc as plsc

_NC, _NS = 2, 16
_NW = _NC * _NS


@functools.partial(jax.jit, static_argnames=("Bb", "Ll", "D", "C"))
def _sc_gather(table, idx, Bb, Ll, D, C):
    B = Bb * Ll
    b_per_w = B // _NW
    rows_per_w = Bb // _NW
    nchunks = b_per_w // C
    bs_per_chunk = C // Ll
    mesh = plsc.VectorSubcoreMesh(core_axis_name="c", subcore_axis_name="s")
    Lp = 56
    Dp = 128

    @functools.partial(
        pl.kernel,
        mesh=mesh,
        out_type=jax.ShapeDtypeStruct((Bb, Lp, Dp), table.dtype),
        compiler_params=pltpu.CompilerParams(use_tc_tiling_on_sc=False),
        scratch_types=[
            pltpu.VMEM((b_per_w,), jnp.int32),
            pltpu.VMEM((C, D), table.dtype),
            pltpu.VMEM((C, D), table.dtype),
            pltpu.SemaphoreType.DMA,
            pltpu.SemaphoreType.DMA,
            pltpu.SemaphoreType.DMA,
            pltpu.SemaphoreType.DMA,
        ],
    )
    def k(table_hbm, idx_hbm, out_hbm, idx_v, rows0, rows1, g0, g1, w0, w1):
        wid = lax.axis_index("s") * _NC + lax.axis_index("c")
        base = wid * b_per_w
        b0 = wid * rows_per_w
        pltpu.sync_copy(idx_hbm.at[pl.ds(base, b_per_w)], idx_v)
        bufs, gs, ws = (rows0, rows1), (g0, g1), (w0, w1)

        def gather_start(b, c):
            pltpu.async_copy(table_hbm.at[idx_v.at[pl.ds(c * C, C)]], bufs[b], gs[b])

        def gather_wait(b):
            pltpu.make_async_copy(
                table_hbm.at[idx_v.at[pl.ds(0, C)]], bufs[b], gs[b]
            ).wait()

        def wb_start(b, c):
            for r in range(bs_per_chunk):
                pltpu.async_copy(
                    bufs[b].at[pl.ds(r * Ll, Ll), :],
                    out_hbm.at[b0 + c * bs_per_chunk + r, pl.ds(0, Ll), pl.ds(0, D)],
                    ws[b],
                )

        def wb_wait(b):
            for r in range(bs_per_chunk):
                pltpu.make_async_copy(
                    bufs[b].at[pl.ds(r * Ll, Ll), :],
                    out_hbm.at[b0, pl.ds(0, Ll), pl.ds(0, D)],
                    ws[b],
                ).wait()

        gather_start(0, 0)
        gather_start(1, 1)

        @pl.loop(0, nchunks - 2, step=2)
        def _(g):
            for b in range(2):
                c = g + b
                gather_wait(b)
                wb_start(b, c)
                wb_wait(b)
                gather_start(b, c + 2)

        for b in range(2):
            gather_wait(b)
            wb_start(b, nchunks - 2 + b)
            wb_wait(b)

    return k(table, idx)


def kernel(input, weight):
    Bb, Ll = input.shape
    D = weight.shape[1]
    B = Bb * Ll
    flat = input.reshape(B).astype(jnp.int32)
    out_p = _sc_gather(weight, flat, Bb=Bb, Ll=Ll, D=D, C=800)
    return out_p[:, :Ll, :D]
